# baseline (speedup 1.0000x reference)
import functools

import jax
import jax.numpy as jnp
from jax import lax
from jax.experimental import pallas as pl
from jax.experimental.pallas import tpu as pltpu
from jax.experimental.pallas import tpu_sc as plsc

_H = 2048
_E = 16
_C = 4
_TEMP = 0.1
_SUB = 256
_NW = 32
_B = 8192
_RPW = _B // _NW


def _score_body(mm_ref, q_ref, w_ref, b_ref, pil_ref, scores_ref):
    p = pil_ref[...].reshape(_C * _E, _H)
    pn = p / jnp.maximum(jnp.sqrt(jnp.sum(p * p, axis=-1, keepdims=True)),
                         1e-12)
    pnb = pn.astype(jnp.bfloat16)
    bias = b_ref[...]

    nt = (((1,), (1,)), ((), ()))
    tb = mm_ref.shape[0]
    for j in range(tb // _SUB):
        sl = pl.ds(j * _SUB, _SUB)
        fb = jnp.concatenate([mm_ref[sl, :].astype(jnp.bfloat16),
                              q_ref[sl, :].astype(jnp.bfloat16)], axis=-1)
        r = jax.lax.dot_general(fb, w_ref[...], nt,
                                preferred_element_type=jnp.float32)
        r = r + bias
        r = r / jnp.maximum(
            jnp.sqrt(jnp.sum(r * r, axis=-1, keepdims=True)), 1e-12)
        simst = jax.lax.dot_general(pnb, r.astype(jnp.bfloat16), nt,
                                    preferred_element_type=jnp.float32)
        scores_ref[:, sl] = (((simst[0:_E, :] + simst[_E:2 * _E, :])
                              + simst[2 * _E:3 * _E, :])
                             + simst[3 * _E:4 * _E, :]) * (1.0 / _C)


def _route_body(scores_hbm, wts_hbm, idx_hbm, probs_hbm,
                st_v, pr_v, wt_v, ix_v):
    wid = lax.axis_index("s") * 2 + lax.axis_index("c")
    base = wid * _RPW
    pltpu.sync_copy(scores_hbm.at[:, pl.ds(base, _RPW)], st_v)
    for g in range(_RPW // 16):
        cols = pl.ds(g * 16, 16)
        v = [st_v[e, cols] * (1.0 / _TEMP) for e in range(_E)]
        m = v[0]
        for e in range(1, _E):
            m = jnp.maximum(m, v[e])
        ex = [jnp.exp(v[e] - m) for e in range(_E)]
        s = ex[0]
        for e in range(1, _E):
            s = s + ex[e]
        prob = [ex[e] / s for e in range(_E)]
        for e in range(_E):
            pr_v[e, cols] = prob[e]
        w1 = prob[0]
        for e in range(1, _E):
            w1 = jnp.maximum(w1, prob[e])
        i1 = jnp.full((16,), _E, jnp.int32)
        for e in range(_E - 1, -1, -1):
            i1 = jnp.where(prob[e] == w1, jnp.int32(e), i1)
        pm = [jnp.where(i1 == e, -jnp.inf, prob[e]) for e in range(_E)]
        w2 = pm[0]
        for e in range(1, _E):
            w2 = jnp.maximum(w2, pm[e])
        i2 = jnp.full((16,), _E, jnp.int32)
        for e in range(_E - 1, -1, -1):
            i2 = jnp.where(pm[e] == w2, jnp.int32(e), i2)
        den = w1 + w2 + 1e-6
        wt_v[0, cols] = w1 / den
        wt_v[1, cols] = w2 / den
        ix_v[0, cols] = i1
        ix_v[1, cols] = i2
    pltpu.sync_copy(pr_v, probs_hbm.at[:, pl.ds(base, _RPW)])
    pltpu.sync_copy(wt_v, wts_hbm.at[:, pl.ds(base, _RPW)])
    pltpu.sync_copy(ix_v, idx_hbm.at[:, pl.ds(base, _RPW)])


def kernel(multimodal_feat, query_feat, pilot_embeddings, W, b):
    bsz, h = multimodal_feat.shape
    tb = min(1024, bsz)
    wt = W.astype(jnp.bfloat16)
    pil = jnp.transpose(pilot_embeddings, (1, 0, 2))
    b2 = b.reshape(1, h)

    scores_t = pl.pallas_call(
        _score_body,
        grid=(bsz // tb,),
        in_specs=[
            pl.BlockSpec((tb, h), lambda i: (i, 0)),
            pl.BlockSpec((tb, h), lambda i: (i, 0)),
            pl.BlockSpec((h, 2 * h), lambda i: (0, 0)),
            pl.BlockSpec((1, h), lambda i: (0, 0)),
            pl.BlockSpec((_C, _E, h), lambda i: (0, 0, 0)),
        ],
        out_specs=pl.BlockSpec((_E, tb), lambda i: (0, i)),
        out_shape=jax.ShapeDtypeStruct((_E, bsz), jnp.float32),
        compiler_params=pltpu.CompilerParams(
            vmem_limit_bytes=100 * 1024 * 1024),
    )(multimodal_feat, query_feat, wt, b2, pil)

    route = functools.partial(
        pl.kernel,
        mesh=plsc.VectorSubcoreMesh(core_axis_name="c", subcore_axis_name="s"),
        out_type=[
            jax.ShapeDtypeStruct((2, bsz), jnp.float32),
            jax.ShapeDtypeStruct((2, bsz), jnp.int32),
            jax.ShapeDtypeStruct((_E, bsz), jnp.float32),
        ],
        scratch_types=[
            pltpu.VMEM((_E, _RPW), jnp.float32),
            pltpu.VMEM((_E, _RPW), jnp.float32),
            pltpu.VMEM((2, _RPW), jnp.float32),
            pltpu.VMEM((2, _RPW), jnp.int32),
        ],
    )(_route_body)
    wts_t, idx_t, probs_t = route(scores_t)

    return (wts_t.T, idx_t.T, probs_t.T)


# speedup vs baseline: 1.0526x; 1.0526x over previous
import functools

import jax
import jax.numpy as jnp
from jax import lax
from jax.experimental import pallas as pl
from jax.experimental.pallas import tpu as pltpu
from jax.experimental.pallas import tpu_sc as plsc

_H = 2048
_E = 16
_C = 4
_TEMP = 0.1
_SUB = 512
_NW = 32
_B = 8192
_RPW = _B // _NW


def _score_body(mm_ref, q_ref, w_ref, b_ref, pil_ref, scores_ref):
    p = pil_ref[...].reshape(_C * _E, _H)
    pn = p / jnp.maximum(jnp.sqrt(jnp.sum(p * p, axis=-1, keepdims=True)),
                         1e-12)
    pnb = pn.astype(jnp.bfloat16)
    bias = b_ref[...]

    nt = (((1,), (1,)), ((), ()))
    tb = mm_ref.shape[0]
    for j in range(tb // _SUB):
        sl = pl.ds(j * _SUB, _SUB)
        fb = jnp.concatenate([mm_ref[sl, :].astype(jnp.bfloat16),
                              q_ref[sl, :].astype(jnp.bfloat16)], axis=-1)
        r = jax.lax.dot_general(fb, w_ref[...], nt,
                                preferred_element_type=jnp.float32)
        r = r + bias
        r = r / jnp.maximum(
            jnp.sqrt(jnp.sum(r * r, axis=-1, keepdims=True)), 1e-12)
        simst = jax.lax.dot_general(pnb, r.astype(jnp.bfloat16), nt,
                                    preferred_element_type=jnp.float32)
        scores_ref[:, sl] = (((simst[0:_E, :] + simst[_E:2 * _E, :])
                              + simst[2 * _E:3 * _E, :])
                             + simst[3 * _E:4 * _E, :]) * (1.0 / _C)


def _route_body(scores_hbm, wts_hbm, idx_hbm, probs_hbm,
                st_v, pr_v, wt_v, ix_v):
    wid = lax.axis_index("s") * 2 + lax.axis_index("c")
    base = wid * _RPW
    pltpu.sync_copy(scores_hbm.at[:, pl.ds(base, _RPW)], st_v)
    for g in range(_RPW // 16):
        cols = pl.ds(g * 16, 16)
        v = [st_v[e, cols] * (1.0 / _TEMP) for e in range(_E)]
        m = v[0]
        for e in range(1, _E):
            m = jnp.maximum(m, v[e])
        ex = [jnp.exp(v[e] - m) for e in range(_E)]
        s = ex[0]
        for e in range(1, _E):
            s = s + ex[e]
        prob = [ex[e] / s for e in range(_E)]
        for e in range(_E):
            pr_v[e, cols] = prob[e]
        w1 = prob[0]
        for e in range(1, _E):
            w1 = jnp.maximum(w1, prob[e])
        i1 = jnp.full((16,), _E, jnp.int32)
        for e in range(_E - 1, -1, -1):
            i1 = jnp.where(prob[e] == w1, jnp.int32(e), i1)
        pm = [jnp.where(i1 == e, -jnp.inf, prob[e]) for e in range(_E)]
        w2 = pm[0]
        for e in range(1, _E):
            w2 = jnp.maximum(w2, pm[e])
        i2 = jnp.full((16,), _E, jnp.int32)
        for e in range(_E - 1, -1, -1):
            i2 = jnp.where(pm[e] == w2, jnp.int32(e), i2)
        den = w1 + w2 + 1e-6
        wt_v[0, cols] = w1 / den
        wt_v[1, cols] = w2 / den
        ix_v[0, cols] = i1
        ix_v[1, cols] = i2
    pltpu.sync_copy(pr_v, probs_hbm.at[:, pl.ds(base, _RPW)])
    pltpu.sync_copy(wt_v, wts_hbm.at[:, pl.ds(base, _RPW)])
    pltpu.sync_copy(ix_v, idx_hbm.at[:, pl.ds(base, _RPW)])


def kernel(multimodal_feat, query_feat, pilot_embeddings, W, b):
    bsz, h = multimodal_feat.shape
    tb = min(1024, bsz)
    wt = W.astype(jnp.bfloat16)
    pil = jnp.transpose(pilot_embeddings, (1, 0, 2))
    b2 = b.reshape(1, h)

    scores_t = pl.pallas_call(
        _score_body,
        grid=(bsz // tb,),
        in_specs=[
            pl.BlockSpec((tb, h), lambda i: (i, 0)),
            pl.BlockSpec((tb, h), lambda i: (i, 0)),
            pl.BlockSpec((h, 2 * h), lambda i: (0, 0)),
            pl.BlockSpec((1, h), lambda i: (0, 0)),
            pl.BlockSpec((_C, _E, h), lambda i: (0, 0, 0)),
        ],
        out_specs=pl.BlockSpec((_E, tb), lambda i: (0, i)),
        out_shape=jax.ShapeDtypeStruct((_E, bsz), jnp.float32),
        compiler_params=pltpu.CompilerParams(
            vmem_limit_bytes=100 * 1024 * 1024),
    )(multimodal_feat, query_feat, wt, b2, pil)

    route = functools.partial(
        pl.kernel,
        mesh=plsc.VectorSubcoreMesh(core_axis_name="c", subcore_axis_name="s"),
        out_type=[
            jax.ShapeDtypeStruct((2, bsz), jnp.float32),
            jax.ShapeDtypeStruct((2, bsz), jnp.int32),
            jax.ShapeDtypeStruct((_E, bsz), jnp.float32),
        ],
        scratch_types=[
            pltpu.VMEM((_E, _RPW), jnp.float32),
            pltpu.VMEM((_E, _RPW), jnp.float32),
            pltpu.VMEM((2, _RPW), jnp.float32),
            pltpu.VMEM((2, _RPW), jnp.int32),
        ],
    )(_route_body)
    wts_t, idx_t, probs_t = route(scores_t)

    return (wts_t.T, idx_t.T, probs_t.T)
